# Initial kernel scaffold; baseline (speedup 1.0000x reference)
import jax
import jax.numpy as jnp
from jax.experimental import pallas as pl


def kernel(x, edge_index, batch, W1, b1, W2, b2, W3, b3, Wl1, bl1, Wl2, bl2):
    raise NotImplementedError("write your pallas kernel here")


# speedup vs baseline: 13.8266x; 13.8266x over previous
import functools

import jax
import jax.numpy as jnp
from jax import lax
from jax.experimental import pallas as pl
from jax.experimental.pallas import tpu as pltpu
from jax.experimental.pallas import tpu_sc as plsc

N = 10000
E = 320000
H = 128
G = 128
OUT = 10

NC = 2
NS = 16
NW = NC * NS
CH = 128
NFULL = (E // NW) // CH
EPW = NFULL * CH
TAIL_BASE = NW * EPW
RPT = 624

_MESH = plsc.VectorSubcoreMesh(core_axis_name="c", subcore_axis_name="s")


def _sc_degree(dst):
    @functools.partial(
        pl.kernel,
        out_type=jax.ShapeDtypeStruct((2 * N, 16), jnp.float32),
        mesh=_MESH,
        scratch_types=[
            pltpu.VMEM((CH,), jnp.int32),
            pltpu.VMEM((CH, 16), jnp.float32),
            pltpu.VMEM((CH, 16), jnp.float32),
            pltpu.VMEM_SHARED((N, 16), jnp.float32),
        ],
    )
    def k(dst_hbm, out_hbm, idx_v, ones_v, zeros_v, acc_sh):
        c = lax.axis_index("c")
        sid = lax.axis_index("s")
        wid = sid * NC + c

        def fill(r, _):
            ones_v[r] = jnp.full((16,), 1.0, jnp.float32)
            zeros_v[r] = jnp.zeros((16,), jnp.float32)
            return 0

        lax.fori_loop(0, CH, fill, 0)
        for kk in range(5):
            pltpu.sync_copy(zeros_v,
                            acc_sh.at[pl.ds(sid * RPT + kk * CH, CH)])
        plsc.subcore_barrier()

        def step(j, _):
            base = pl.multiple_of(wid * EPW + j * CH, CH)
            pltpu.sync_copy(dst_hbm.at[pl.ds(base, CH)], idx_v)
            pltpu.sync_copy(ones_v, acc_sh.at[idx_v], add=True)
            return 0

        lax.fori_loop(0, NFULL, step, 0)

        @pl.when(wid < 4)
        def _():
            base = pl.multiple_of(TAIL_BASE + wid * CH, CH)
            pltpu.sync_copy(dst_hbm.at[pl.ds(base, CH)], idx_v)
            pltpu.sync_copy(ones_v, acc_sh.at[idx_v], add=True)

        plsc.subcore_barrier()
        pltpu.sync_copy(
            acc_sh.at[pl.ds(sid * RPT, RPT)],
            out_hbm.at[pl.ds(c * N + sid * RPT, RPT)])

        @pl.when(sid == NS - 1)
        def _():
            pltpu.sync_copy(acc_sh.at[pl.ds(NS * RPT, N - NS * RPT)],
                            out_hbm.at[pl.ds(c * N + NS * RPT, N - NS * RPT)])

    return k(dst)


def _sc_propagate(s, src, dst):
    @functools.partial(
        pl.kernel,
        out_type=jax.ShapeDtypeStruct((2 * N, H), jnp.float32),
        mesh=_MESH,
        scratch_types=[
            pltpu.VMEM((CH,), jnp.int32),
            pltpu.VMEM((CH,), jnp.int32),
            pltpu.VMEM((CH, H), jnp.float32),
            pltpu.VMEM_SHARED((N, H), jnp.float32),
            pltpu.SemaphoreType.DMA,
        ],
    )
    def k(s_hbm, src_hbm, dst_hbm, out_hbm, isrc_v, idst_v, rows_v, acc_sh, sem):
        c = lax.axis_index("c")
        sid = lax.axis_index("s")
        wid = sid * NC + c

        def zrow(r, _):
            for kk in range(H // 16):
                rows_v[r, pl.ds(16 * kk, 16)] = jnp.zeros((16,), jnp.float32)
            return 0

        lax.fori_loop(0, CH, zrow, 0)
        for kk in range(5):
            pltpu.sync_copy(rows_v,
                            acc_sh.at[pl.ds(sid * RPT + kk * CH, CH)])
        plsc.subcore_barrier()

        def step(j, _):
            base = pl.multiple_of(wid * EPW + j * CH, CH)
            pltpu.sync_copy(src_hbm.at[pl.ds(base, CH)], isrc_v)
            pltpu.sync_copy(dst_hbm.at[pl.ds(base, CH)], idst_v)
            pltpu.async_copy(s_hbm.at[isrc_v], rows_v, sem).wait()
            pltpu.sync_copy(rows_v, acc_sh.at[idst_v], add=True)
            return 0

        lax.fori_loop(0, NFULL, step, 0)

        @pl.when(wid < 4)
        def _():
            base = pl.multiple_of(TAIL_BASE + wid * CH, CH)
            pltpu.sync_copy(src_hbm.at[pl.ds(base, CH)], isrc_v)
            pltpu.sync_copy(dst_hbm.at[pl.ds(base, CH)], idst_v)
            pltpu.async_copy(s_hbm.at[isrc_v], rows_v, sem).wait()
            pltpu.sync_copy(rows_v, acc_sh.at[idst_v], add=True)

        plsc.subcore_barrier()
        pltpu.sync_copy(
            acc_sh.at[pl.ds(sid * RPT, RPT)],
            out_hbm.at[pl.ds(c * N + sid * RPT, RPT)])

        @pl.when(sid == NS - 1)
        def _():
            pltpu.sync_copy(acc_sh.at[pl.ds(NS * RPT, N - NS * RPT)],
                            out_hbm.at[pl.ds(c * N + NS * RPT, N - NS * RPT)])

    return k(s, src, dst)


R = 1000


def _tc_first(x, W1, degp):
    def body(x_ref, w_ref, dg_ref, lin_ref, s_ref, dis_ref, dinv_ref):
        deg = 1.0 + dg_ref[0, :, 0:1] + dg_ref[1, :, 0:1]
        dis = lax.rsqrt(deg)
        dinv = 1.0 / deg
        lin = jnp.dot(x_ref[...], w_ref[...], preferred_element_type=jnp.float32)
        lin_ref[...] = lin
        s_ref[...] = lin * dis
        dis_ref[...] = jnp.broadcast_to(dis, (R, H))
        dinv_ref[...] = jnp.broadcast_to(dinv, (R, H))

    o = jax.ShapeDtypeStruct((N, H), jnp.float32)
    return pl.pallas_call(
        body,
        grid=(N // R,),
        in_specs=[
            pl.BlockSpec((R, H), lambda i: (i, 0)),
            pl.BlockSpec((H, H), lambda i: (0, 0)),
            pl.BlockSpec((2, R, 16), lambda i: (0, i, 0)),
        ],
        out_specs=[pl.BlockSpec((R, H), lambda i: (i, 0))] * 4,
        out_shape=[o, o, o, o],
    )(x, W1, degp)


def _tc_layer(accp, lin, disb, dinvb, b, Wn):
    def body(a_ref, lin_ref, dis_ref, dinv_ref, b_ref, w_ref, lin2_ref, s2_ref):
        acc = a_ref[0] + a_ref[1]
        h = jnp.maximum(
            acc * dis_ref[...] + lin_ref[...] * dinv_ref[...] + b_ref[...], 0.0)
        lin2 = jnp.dot(h, w_ref[...], preferred_element_type=jnp.float32)
        lin2_ref[...] = lin2
        s2_ref[...] = lin2 * dis_ref[...]

    o = jax.ShapeDtypeStruct((N, H), jnp.float32)
    return pl.pallas_call(
        body,
        grid=(N // R,),
        in_specs=[
            pl.BlockSpec((2, R, H), lambda i: (0, i, 0)),
            pl.BlockSpec((R, H), lambda i: (i, 0)),
            pl.BlockSpec((R, H), lambda i: (i, 0)),
            pl.BlockSpec((R, H), lambda i: (i, 0)),
            pl.BlockSpec((1, H), lambda i: (0, 0)),
            pl.BlockSpec((H, H), lambda i: (0, 0)),
        ],
        out_specs=[pl.BlockSpec((R, H), lambda i: (i, 0))] * 2,
        out_shape=[o, o],
    )(accp, lin, disb, dinvb, b, Wn)


def _tc_pool(accp, lin, disb, dinvb, b, batchb):
    def body(a_ref, lin_ref, dis_ref, dinv_ref, b_ref, bat_ref, seg_ref, cnt_ref):
        acc = a_ref[0] + a_ref[1]
        h = jnp.maximum(
            acc * dis_ref[...] + lin_ref[...] * dinv_ref[...] + b_ref[...], 0.0)
        gid = lax.broadcasted_iota(jnp.int32, (R, G), 1)
        onehot = jnp.where(bat_ref[...] == gid, 1.0, 0.0)
        segc = lax.dot_general(onehot, h, (((0,), (0,)), ((), ())),
                               preferred_element_type=jnp.float32)
        cntc = lax.dot_general(onehot, jnp.ones((R, H), jnp.float32),
                               (((0,), (0,)), ((), ())),
                               preferred_element_type=jnp.float32)

        @pl.when(pl.program_id(0) == 0)
        def _():
            seg_ref[...] = segc
            cnt_ref[...] = cntc

        @pl.when(pl.program_id(0) != 0)
        def _():
            seg_ref[...] += segc
            cnt_ref[...] += cntc

    o = jax.ShapeDtypeStruct((G, H), jnp.float32)
    return pl.pallas_call(
        body,
        grid=(N // R,),
        in_specs=[
            pl.BlockSpec((2, R, H), lambda i: (0, i, 0)),
            pl.BlockSpec((R, H), lambda i: (i, 0)),
            pl.BlockSpec((R, H), lambda i: (i, 0)),
            pl.BlockSpec((R, H), lambda i: (i, 0)),
            pl.BlockSpec((1, H), lambda i: (0, 0)),
            pl.BlockSpec((R, G), lambda i: (i, 0)),
        ],
        out_specs=[pl.BlockSpec((G, H), lambda i: (0, 0))] * 2,
        out_shape=[o, o],
    )(accp, lin, disb, dinvb, b, batchb)


def _tc_head(seg, cnt, Wl1, bl1, Wl2p, bl2p):
    def body(seg_ref, cnt_ref, w1_ref, b1_ref, w2_ref, b2_ref, out_ref):
        pooled = seg_ref[...] / jnp.maximum(cnt_ref[...], 1.0)
        z = jnp.maximum(
            jnp.dot(pooled, w1_ref[...], preferred_element_type=jnp.float32)
            + b1_ref[...], 0.0)
        out_ref[...] = (
            jnp.dot(z, w2_ref[...], preferred_element_type=jnp.float32)
            + b2_ref[...])

    return pl.pallas_call(
        body,
        out_shape=jax.ShapeDtypeStruct((G, H), jnp.float32),
    )(seg, cnt, Wl1, bl1, Wl2p, bl2p)


def kernel(x, edge_index, batch, W1, b1, W2, b2, W3, b3, Wl1, bl1, Wl2, bl2):
    src = edge_index[0]
    dst = edge_index[1]

    degp = _sc_degree(dst).reshape(2, N, 16)
    lin1, s1, disb, dinvb = _tc_first(x, W1, degp)

    acc1 = _sc_propagate(s1, src, dst).reshape(2, N, H)
    lin2, s2 = _tc_layer(acc1, lin1, disb, dinvb, b1.reshape(1, H), W2)

    acc2 = _sc_propagate(s2, src, dst).reshape(2, N, H)
    lin3, s3 = _tc_layer(acc2, lin2, disb, dinvb, b2.reshape(1, H), W3)

    acc3 = _sc_propagate(s3, src, dst).reshape(2, N, H)
    batchb = jnp.broadcast_to(batch[:, None], (N, G))
    seg, cnt = _tc_pool(acc3, lin3, disb, dinvb, b3.reshape(1, H), batchb)

    Wl2p = jnp.pad(Wl2, ((0, 0), (0, H - OUT)))
    bl2p = jnp.pad(bl2, (0, H - OUT)).reshape(1, H)
    outp = _tc_head(seg, cnt, Wl1, bl1.reshape(1, H), Wl2p, bl2p)
    return outp[:, :OUT]
